# baseline (speedup 1.0000x reference)
import functools

import jax
import jax.numpy as jnp
from jax import lax
from jax.experimental import pallas as pl
from jax.experimental.pallas import tpu as pltpu
from jax.experimental.pallas import tpu_sc as plsc

VOCAB = 1000000
DIM = 32
B = 16384
ALPHA = 0.75
X_MAX = 100.0

_NC = 2
_NS = 16
_NW = _NC * _NS
_BPW = B // _NW


def _sc_gather_body(i_hbm, j_hbm, w_hbm, c_hbm, wb_hbm, cb_hbm,
                    w_out, c_out, wb_out, cb_out,
                    idx_i, idx_j, w_v, c_v, wb_v, cb_v, sem):
    wid = lax.axis_index("s") * _NC + lax.axis_index("c")
    base = wid * _BPW
    pltpu.sync_copy(i_hbm.at[pl.ds(base, _BPW)], idx_i)
    pltpu.sync_copy(j_hbm.at[pl.ds(base, _BPW)], idx_j)
    cp1 = pltpu.async_copy(w_hbm.at[idx_i], w_v, sem)
    cp2 = pltpu.async_copy(c_hbm.at[idx_j], c_v, sem)
    cp3 = pltpu.async_copy(wb_hbm.at[idx_i], wb_v, sem)
    cp4 = pltpu.async_copy(cb_hbm.at[idx_j], cb_v, sem)
    cp1.wait()
    cp2.wait()
    cp3.wait()
    cp4.wait()
    pltpu.sync_copy(w_v, w_out.at[pl.ds(base, _BPW)])
    pltpu.sync_copy(c_v, c_out.at[pl.ds(base, _BPW)])
    pltpu.sync_copy(wb_v, wb_out.at[pl.ds(base, _BPW)])
    pltpu.sync_copy(cb_v, cb_out.at[pl.ds(base, _BPW)])


_sc_gather = functools.partial(
    pl.kernel,
    mesh=plsc.VectorSubcoreMesh(core_axis_name="c", subcore_axis_name="s"),
    out_type=[
        jax.ShapeDtypeStruct((B, DIM), jnp.bfloat16),
        jax.ShapeDtypeStruct((B, DIM), jnp.bfloat16),
        jax.ShapeDtypeStruct((B,), jnp.float32),
        jax.ShapeDtypeStruct((B,), jnp.float32),
    ],
    scratch_types=[
        pltpu.VMEM((_BPW,), jnp.int32),
        pltpu.VMEM((_BPW,), jnp.int32),
        pltpu.VMEM((_BPW, DIM), jnp.bfloat16),
        pltpu.VMEM((_BPW, DIM), jnp.bfloat16),
        pltpu.VMEM((_BPW,), jnp.float32),
        pltpu.VMEM((_BPW,), jnp.float32),
        pltpu.SemaphoreType.DMA,
    ],
    compiler_params=pltpu.CompilerParams(use_tc_tiling_on_sc=False),
)(_sc_gather_body)


_TC_BLK = 2048


def _tc_loss_body(x_ref, w_ref, c_ref, wb_ref, cb_ref, o_ref):
    w = w_ref[...].astype(jnp.float32)
    c = c_ref[...].astype(jnp.float32)
    s = jnp.sum(w * c, axis=1)
    x = x_ref[...]
    f = jnp.where(x < X_MAX, (x * (1.0 / X_MAX)) ** ALPHA, jnp.float32(1.0))
    o_ref[...] = f * (s + wb_ref[...] + cb_ref[...] - jnp.log(x))


def _tc_loss(x, w_rows, c_rows, wb, cb):
    grid = (B // _TC_BLK,)
    vec_spec = pl.BlockSpec((_TC_BLK,), lambda k: (k,))
    row_spec = pl.BlockSpec((_TC_BLK, DIM), lambda k: (k, 0))
    return pl.pallas_call(
        _tc_loss_body,
        grid=grid,
        in_specs=[vec_spec, row_spec, row_spec, vec_spec, vec_spec],
        out_specs=vec_spec,
        out_shape=jax.ShapeDtypeStruct((B,), jnp.float32),
    )(x, w_rows, c_rows, wb, cb)


def kernel(x, i, j, w_table, c_table, w_bias, c_bias):
    i32 = i.astype(jnp.int32)
    j32 = j.astype(jnp.int32)
    w_rows, c_rows, wb, cb = _sc_gather(
        i32, j32, w_table.astype(jnp.bfloat16), c_table.astype(jnp.bfloat16),
        w_bias, c_bias)
    return _tc_loss(x, w_rows, c_rows, wb, cb)


# speedup vs baseline: 1.0450x; 1.0450x over previous
import functools

import jax
import jax.numpy as jnp
from jax import lax
from jax.experimental import pallas as pl
from jax.experimental.pallas import tpu as pltpu
from jax.experimental.pallas import tpu_sc as plsc

VOCAB = 1000000
DIM = 32
B = 16384
ALPHA = 0.75
X_MAX = 100.0

_NC = 2
_NS = 16
_NW = _NC * _NS
_BPW = B // _NW
_PH = 128
_L = 16


def _sc_rows_body(i_hbm, j_hbm, w2_hbm, c2_hbm,
                  w_out, c_out,
                  idx_i, idx_j, qi, qj, buf, sem):
    wid = lax.axis_index("s") * _NC + lax.axis_index("c")
    base = wid * _BPW
    pltpu.sync_copy(i_hbm.at[pl.ds(base, _BPW)], idx_i)
    pltpu.sync_copy(j_hbm.at[pl.ds(base, _BPW)], idx_j)
    for g in range(_BPW // _L):
        o = g * _L
        qi[pl.ds(o, _L)] = lax.shift_right_logical(idx_i[pl.ds(o, _L)], 2)
        qj[pl.ds(o, _L)] = lax.shift_right_logical(idx_j[pl.ds(o, _L)], 2)
    for p in range(_BPW // _PH):
        po = p * _PH
        pltpu.async_copy(w2_hbm.at[qi.at[pl.ds(po, _PH)]], buf, sem).wait()
        pltpu.sync_copy(buf, w_out.at[pl.ds(base + po, _PH), :])
    for p in range(_BPW // _PH):
        po = p * _PH
        pltpu.async_copy(c2_hbm.at[qj.at[pl.ds(po, _PH)]], buf, sem).wait()
        pltpu.sync_copy(buf, c_out.at[pl.ds(base + po, _PH), :])


_sc_rows = functools.partial(
    pl.kernel,
    mesh=plsc.VectorSubcoreMesh(core_axis_name="c", subcore_axis_name="s"),
    out_type=[
        jax.ShapeDtypeStruct((B, 4 * DIM), jnp.float32),
        jax.ShapeDtypeStruct((B, 4 * DIM), jnp.float32),
    ],
    scratch_types=[
        pltpu.VMEM((_BPW,), jnp.int32),
        pltpu.VMEM((_BPW,), jnp.int32),
        pltpu.VMEM((_BPW,), jnp.int32),
        pltpu.VMEM((_BPW,), jnp.int32),
        pltpu.VMEM((_PH, 4 * DIM), jnp.float32),
        pltpu.SemaphoreType.DMA,
    ],
)(_sc_rows_body)


def _sc_bias_body(i_hbm, j_hbm, wb_hbm, cb_hbm,
                  wb_out, cb_out,
                  idx_i, idx_j, wb_v, cb_v, sem):
    wid = lax.axis_index("s") * _NC + lax.axis_index("c")
    base = wid * _BPW
    pltpu.sync_copy(i_hbm.at[pl.ds(base, _BPW)], idx_i)
    pltpu.sync_copy(j_hbm.at[pl.ds(base, _BPW)], idx_j)
    cp1 = pltpu.async_copy(wb_hbm.at[idx_i], wb_v, sem)
    cp2 = pltpu.async_copy(cb_hbm.at[idx_j], cb_v, sem)
    cp1.wait()
    cp2.wait()
    pltpu.sync_copy(wb_v, wb_out.at[pl.ds(base, _BPW)])
    pltpu.sync_copy(cb_v, cb_out.at[pl.ds(base, _BPW)])


_sc_bias = functools.partial(
    pl.kernel,
    mesh=plsc.VectorSubcoreMesh(core_axis_name="c", subcore_axis_name="s"),
    out_type=[
        jax.ShapeDtypeStruct((B,), jnp.float32),
        jax.ShapeDtypeStruct((B,), jnp.float32),
    ],
    scratch_types=[
        pltpu.VMEM((_BPW,), jnp.int32),
        pltpu.VMEM((_BPW,), jnp.int32),
        pltpu.VMEM((_BPW,), jnp.float32),
        pltpu.VMEM((_BPW,), jnp.float32),
        pltpu.SemaphoreType.DMA,
    ],
    compiler_params=pltpu.CompilerParams(use_tc_tiling_on_sc=False),
)(_sc_bias_body)


_TC_BLK = 2048


def _tc_loss_body(x_ref, i_ref, j_ref, w_ref, c_ref, wb_ref, cb_ref, o_ref):
    w = w_ref[...]
    c = c_ref[...]
    qa = jnp.bitwise_and(i_ref[...], 3)
    qc = jnp.bitwise_and(j_ref[...], 3)
    s = jnp.zeros((_TC_BLK,), jnp.float32)
    for a in range(4):
        wa = w[:, a * DIM:(a + 1) * DIM]
        sel_a = qa == a
        for b in range(4):
            d = jnp.sum(wa * c[:, b * DIM:(b + 1) * DIM], axis=1)
            s = jnp.where(jnp.logical_and(sel_a, qc == b), d, s)
    x = x_ref[...]
    f = jnp.where(x < X_MAX, (x * (1.0 / X_MAX)) ** ALPHA, jnp.float32(1.0))
    o_ref[...] = f * (s + wb_ref[...] + cb_ref[...] - jnp.log(x))


def _tc_loss(x, i32, j32, w128, c128, wb, cb):
    vec_spec = pl.BlockSpec((_TC_BLK,), lambda k: (k,))
    mat_spec = pl.BlockSpec((_TC_BLK, 4 * DIM), lambda k: (k, 0))
    return pl.pallas_call(
        _tc_loss_body,
        grid=(B // _TC_BLK,),
        in_specs=[vec_spec, vec_spec, vec_spec, mat_spec, mat_spec,
                  vec_spec, vec_spec],
        out_specs=vec_spec,
        out_shape=jax.ShapeDtypeStruct((B,), jnp.float32),
    )(x, i32, j32, w128, c128, wb, cb)


def kernel(x, i, j, w_table, c_table, w_bias, c_bias):
    i32 = i.astype(jnp.int32)
    j32 = j.astype(jnp.int32)
    w2 = w_table.reshape(VOCAB // 4, 4 * DIM)
    c2 = c_table.reshape(VOCAB // 4, 4 * DIM)
    w128, c128 = _sc_rows(i32, j32, w2, c2)
    wb, cb = _sc_bias(i32, j32, w_bias, c_bias)
    return _tc_loss(x, i32, j32, w128, c128, wb, cb)


# speedup vs baseline: 1.1618x; 1.1118x over previous
import functools

import jax
import jax.numpy as jnp
from jax import lax
from jax.experimental import pallas as pl
from jax.experimental.pallas import tpu as pltpu
from jax.experimental.pallas import tpu_sc as plsc

VOCAB = 1000000
DIM = 32
B = 16384
ALPHA = 0.75
X_MAX = 100.0

_NC = 2
_NS = 16
_NW = _NC * _NS
_BPW = B // _NW
_PH = 128
_PW = 128


def _sc_rows_body(i_hbm, j_hbm, wp_hbm, cp_hbm,
                  w_out, c_out,
                  idx_i, idx_j, buf_a, buf_b, sem_a, sem_b):
    wid = lax.axis_index("s") * _NC + lax.axis_index("c")
    base = wid * _BPW
    pltpu.sync_copy(i_hbm.at[pl.ds(base, _BPW)], idx_i)
    pltpu.sync_copy(j_hbm.at[pl.ds(base, _BPW)], idx_j)
    n_ph = _BPW // _PH
    specs = [(wp_hbm, idx_i, w_out, p) for p in range(n_ph)]
    specs += [(cp_hbm, idx_j, c_out, p) for p in range(n_ph)]
    bufs = (buf_a, buf_b)
    sems = (sem_a, sem_b)
    copies = []
    for n, (tab, idx, _, p) in enumerate(specs):
        copies.append(pltpu.async_copy(
            tab.at[idx.at[pl.ds(p * _PH, _PH)]], bufs[n % 2], sems[n % 2]))
        if n >= 1:
            _, _, out_p, p_p = specs[n - 1]
            copies[n - 1].wait()
            pltpu.sync_copy(bufs[(n - 1) % 2],
                            out_p.at[pl.ds(base + p_p * _PH, _PH), :])
    n = len(specs) - 1
    _, _, out_l, p_l = specs[n]
    copies[n].wait()
    pltpu.sync_copy(bufs[n % 2], out_l.at[pl.ds(base + p_l * _PH, _PH), :])


_sc_rows = functools.partial(
    pl.kernel,
    mesh=plsc.VectorSubcoreMesh(core_axis_name="c", subcore_axis_name="s"),
    out_type=[
        jax.ShapeDtypeStruct((B, _PW), jnp.float32),
        jax.ShapeDtypeStruct((B, _PW), jnp.float32),
    ],
    scratch_types=[
        pltpu.VMEM((_BPW,), jnp.int32),
        pltpu.VMEM((_BPW,), jnp.int32),
        pltpu.VMEM((_PH, _PW), jnp.float32),
        pltpu.VMEM((_PH, _PW), jnp.float32),
        pltpu.SemaphoreType.DMA,
        pltpu.SemaphoreType.DMA,
    ],
)(_sc_rows_body)


def _sc_bias_body(i_hbm, j_hbm, wb_hbm, cb_hbm,
                  wb_out, cb_out,
                  idx_i, idx_j, wb_v, cb_v, sem):
    wid = lax.axis_index("s") * _NC + lax.axis_index("c")
    base = wid * _BPW
    pltpu.sync_copy(i_hbm.at[pl.ds(base, _BPW)], idx_i)
    pltpu.sync_copy(j_hbm.at[pl.ds(base, _BPW)], idx_j)
    cp1 = pltpu.async_copy(wb_hbm.at[idx_i], wb_v, sem)
    cp2 = pltpu.async_copy(cb_hbm.at[idx_j], cb_v, sem)
    cp1.wait()
    cp2.wait()
    pltpu.sync_copy(wb_v, wb_out.at[pl.ds(base, _BPW)])
    pltpu.sync_copy(cb_v, cb_out.at[pl.ds(base, _BPW)])


_sc_bias = functools.partial(
    pl.kernel,
    mesh=plsc.VectorSubcoreMesh(core_axis_name="c", subcore_axis_name="s"),
    out_type=[
        jax.ShapeDtypeStruct((B,), jnp.float32),
        jax.ShapeDtypeStruct((B,), jnp.float32),
    ],
    scratch_types=[
        pltpu.VMEM((_BPW,), jnp.int32),
        pltpu.VMEM((_BPW,), jnp.int32),
        pltpu.VMEM((_BPW,), jnp.float32),
        pltpu.VMEM((_BPW,), jnp.float32),
        pltpu.SemaphoreType.DMA,
    ],
    compiler_params=pltpu.CompilerParams(use_tc_tiling_on_sc=False),
)(_sc_bias_body)


_TC_BLK = 2048


def _tc_loss_body(x_ref, w_ref, c_ref, wb_ref, cb_ref, o_ref):
    s = jnp.sum(w_ref[:, :DIM] * c_ref[:, :DIM], axis=1)
    x = x_ref[...]
    f = jnp.where(x < X_MAX, (x * (1.0 / X_MAX)) ** ALPHA, jnp.float32(1.0))
    o_ref[...] = f * (s + wb_ref[...] + cb_ref[...] - jnp.log(x))


def _tc_loss(x, w128, c128, wb, cb):
    vec_spec = pl.BlockSpec((_TC_BLK,), lambda k: (k,))
    mat_spec = pl.BlockSpec((_TC_BLK, _PW), lambda k: (k, 0))
    return pl.pallas_call(
        _tc_loss_body,
        grid=(B // _TC_BLK,),
        in_specs=[vec_spec, mat_spec, mat_spec, vec_spec, vec_spec],
        out_specs=vec_spec,
        out_shape=jax.ShapeDtypeStruct((B,), jnp.float32),
    )(x, w128, c128, wb, cb)


def kernel(x, i, j, w_table, c_table, w_bias, c_bias):
    i32 = i.astype(jnp.int32)
    j32 = j.astype(jnp.int32)
    wp = jnp.pad(w_table, ((0, 0), (0, _PW - DIM)))
    cp = jnp.pad(c_table, ((0, 0), (0, _PW - DIM)))
    w128, c128 = _sc_rows(i32, j32, wp, cp)
    wb, cb = _sc_bias(i32, j32, w_bias, c_bias)
    return _tc_loss(x, w128, c128, wb, cb)


# speedup vs baseline: 1.8116x; 1.5593x over previous
import functools

import jax
import jax.numpy as jnp
from jax import lax
from jax.experimental import pallas as pl
from jax.experimental.pallas import tpu as pltpu
from jax.experimental.pallas import tpu_sc as plsc

VOCAB = 1000000
DIM = 32
B = 16384
ALPHA = 0.75
X_MAX = 100.0

_NC = 2
_NS = 16
_NW = _NC * _NS
_BPW = B // _NW
_L = 16

_CH = 512
_NCHUNK = 999936 // _CH
_TAIL0 = _NCHUNK * _CH
_NTAIL = VOCAB - _TAIL0
_CW = DIM * _CH
_TAILPOS = _NCHUNK * _CW
_OUTN = _TAILPOS + DIM * _NTAIL


def _sc_relayout_body(wt_hbm, ct_hbm, tw_hbm, tc_hbm,
                      ow_hbm, oc_hbm,
                      buf_w, buf_c, tbuf, sem):
    wid = lax.axis_index("s") * _NC + lax.axis_index("c")

    def step(k, _):
        c = wid + k * _NW

        @pl.when(c < _NCHUNK)
        def _():
            co = pl.multiple_of(c * _CH, _CH)
            do = pl.multiple_of(c * _CW, _CH)
            pltpu.sync_copy(wt_hbm.at[:, pl.ds(co, _CH)], buf_w)
            pltpu.sync_copy(ct_hbm.at[:, pl.ds(co, _CH)], buf_c)
            for d in range(DIM):
                pltpu.sync_copy(
                    buf_w.at[d, :],
                    ow_hbm.at[pl.ds(do + d * _CH, _CH)])
                pltpu.sync_copy(
                    buf_c.at[d, :],
                    oc_hbm.at[pl.ds(do + d * _CH, _CH)])
        return ()

    lax.fori_loop(0, (_NCHUNK + _NW - 1) // _NW, step, ())

    @pl.when(wid == 0)
    def _():
        pltpu.sync_copy(tw_hbm, tbuf)
        pltpu.sync_copy(tbuf, ow_hbm.at[pl.ds(_TAILPOS, DIM * _NTAIL)])
        pltpu.sync_copy(tc_hbm, tbuf)
        pltpu.sync_copy(tbuf, oc_hbm.at[pl.ds(_TAILPOS, DIM * _NTAIL)])


_sc_relayout = functools.partial(
    pl.kernel,
    mesh=plsc.VectorSubcoreMesh(core_axis_name="c", subcore_axis_name="s"),
    out_type=[
        jax.ShapeDtypeStruct((_OUTN,), jnp.float32),
        jax.ShapeDtypeStruct((_OUTN,), jnp.float32),
    ],
    scratch_types=[
        pltpu.VMEM((DIM, _CH), jnp.float32),
        pltpu.VMEM((DIM, _CH), jnp.float32),
        pltpu.VMEM((DIM * _NTAIL,), jnp.float32),
        pltpu.SemaphoreType.DMA,
    ],
)(_sc_relayout_body)


def _sc_dot_body(i_hbm, j_hbm, ow_hbm, oc_hbm, wb_hbm, cb_hbm,
                 s_out,
                 idx_i, idx_j, base_i, step_i, base_j, step_j,
                 addr_w, addr_c, w_v, c_v, wb_v, cb_v, out_v,
                 sem_w, sem_c, sem_b):
    wid = lax.axis_index("s") * _NC + lax.axis_index("c")
    base = wid * _BPW
    pltpu.sync_copy(i_hbm.at[pl.ds(base, _BPW)], idx_i)
    pltpu.sync_copy(j_hbm.at[pl.ds(base, _BPW)], idx_j)

    cpb1 = pltpu.async_copy(wb_hbm.at[idx_i], wb_v, sem_b)
    cpb2 = pltpu.async_copy(cb_hbm.at[idx_j], cb_v, sem_b)

    def addr_prep(g, _):
        o = g * _L
        for idx, bs, st in ((idx_i, base_i, step_i),
                            (idx_j, base_j, step_j)):
            r = idx[pl.ds(o, _L)]
            tail = r >= _TAIL0
            main_b = ((lax.shift_right_logical(r, 9) * _CW)
                      + jnp.bitwise_and(r, _CH - 1))
            tail_b = _TAILPOS + (r - _TAIL0)
            bs[pl.ds(o, _L)] = jnp.where(tail, tail_b, main_b)
            st[pl.ds(o, _L)] = jnp.where(tail, _NTAIL, _CH)
        return ()

    lax.fori_loop(0, _BPW // _L, addr_prep, ())

    for d in range(DIM):
        def addr_fill(g, _):
            o = g * _L
            addr_w[pl.ds(o, _L)] = base_i[pl.ds(o, _L)] + d * step_i[pl.ds(o, _L)]
            addr_c[pl.ds(o, _L)] = base_j[pl.ds(o, _L)] + d * step_j[pl.ds(o, _L)]
            return ()

        lax.fori_loop(0, _BPW // _L, addr_fill, ())
        pltpu.async_copy(ow_hbm.at[addr_w], w_v.at[d, :], sem_w).wait()
        pltpu.async_copy(oc_hbm.at[addr_c], c_v.at[d, :], sem_c).wait()

    cpb1.wait()
    cpb2.wait()

    def dot_group(g, _):
        o = g * _L
        acc = wb_v[pl.ds(o, _L)] + cb_v[pl.ds(o, _L)]
        for d in range(DIM):
            acc = acc + w_v[d, pl.ds(o, _L)] * c_v[d, pl.ds(o, _L)]
        out_v[pl.ds(o, _L)] = acc
        return ()

    lax.fori_loop(0, _BPW // _L, dot_group, ())
    pltpu.sync_copy(out_v, s_out.at[pl.ds(base, _BPW)])


_sc_dot = functools.partial(
    pl.kernel,
    mesh=plsc.VectorSubcoreMesh(core_axis_name="c", subcore_axis_name="s"),
    out_type=jax.ShapeDtypeStruct((B,), jnp.float32),
    scratch_types=[
        pltpu.VMEM((_BPW,), jnp.int32),
        pltpu.VMEM((_BPW,), jnp.int32),
        pltpu.VMEM((_BPW,), jnp.int32),
        pltpu.VMEM((_BPW,), jnp.int32),
        pltpu.VMEM((_BPW,), jnp.int32),
        pltpu.VMEM((_BPW,), jnp.int32),
        pltpu.VMEM((_BPW,), jnp.int32),
        pltpu.VMEM((_BPW,), jnp.int32),
        pltpu.VMEM((DIM, _BPW), jnp.float32),
        pltpu.VMEM((DIM, _BPW), jnp.float32),
        pltpu.VMEM((_BPW,), jnp.float32),
        pltpu.VMEM((_BPW,), jnp.float32),
        pltpu.VMEM((_BPW,), jnp.float32),
        pltpu.SemaphoreType.DMA,
        pltpu.SemaphoreType.DMA,
        pltpu.SemaphoreType.DMA,
    ],
    compiler_params=pltpu.CompilerParams(use_tc_tiling_on_sc=False),
)(_sc_dot_body)


_TC_BLK = 4096


def _tc_loss_body(x_ref, s_ref, o_ref):
    x = x_ref[...]
    f = jnp.where(x < X_MAX, (x * (1.0 / X_MAX)) ** ALPHA, jnp.float32(1.0))
    o_ref[...] = f * (s_ref[...] - jnp.log(x))


def _tc_loss(x, s):
    vec_spec = pl.BlockSpec((_TC_BLK,), lambda k: (k,))
    return pl.pallas_call(
        _tc_loss_body,
        grid=(B // _TC_BLK,),
        in_specs=[vec_spec, vec_spec],
        out_specs=vec_spec,
        out_shape=jax.ShapeDtypeStruct((B,), jnp.float32),
    )(x, s)


def kernel(x, i, j, w_table, c_table, w_bias, c_bias):
    i32 = i.astype(jnp.int32)
    j32 = j.astype(jnp.int32)
    tw = w_table[_TAIL0:].T.reshape(-1)
    tc = c_table[_TAIL0:].T.reshape(-1)
    ow, oc = _sc_relayout(w_table.T, c_table.T, tw, tc)
    s = _sc_dot(i32, j32, ow, oc, w_bias, c_bias)
    return _tc_loss(x, s)


# speedup vs baseline: 3.2562x; 1.7974x over previous
import functools

import jax
import jax.numpy as jnp
from jax import lax
from jax.experimental import pallas as pl
from jax.experimental.pallas import tpu as pltpu
from jax.experimental.pallas import tpu_sc as plsc

VOCAB = 1000000
DIM = 32
B = 16384
ALPHA = 0.75
X_MAX = 100.0

_NC = 2
_NS = 16
_NW = _NC * _NS
_BPW = B // _NW
_L = 16

_CH = 512
_NCHUNK = 999936 // _CH
_TAIL0 = _NCHUNK * _CH
_NTAIL = VOCAB - _TAIL0
_CW = DIM * _CH
_TAILPOS = _NCHUNK * _CW
_OUTN = _TAILPOS + DIM * _NTAIL


def _sc_relayout_body(wt_hbm, ct_hbm, tw_hbm, tc_hbm,
                      ow_hbm, oc_hbm,
                      buf_w0, buf_c0, buf_w1, buf_c1, tbuf, sem_out):
    wid = lax.axis_index("s") * _NC + lax.axis_index("c")
    n_iter = (_NCHUNK + _NW - 1) // _NW
    bufs = ((buf_w0, buf_c0), (buf_w1, buf_c1))

    def step_slot(k, slot):
        bw, bc = bufs[slot]
        c = wid + k * _NW

        @pl.when(k >= 2)
        def _():
            c2 = wid + (k - 2) * _NW

            @pl.when(c2 < _NCHUNK)
            def _():
                pltpu.make_async_copy(
                    wt_hbm.at[:, pl.ds(0, _CH)], bw, sem_out).wait()
                pltpu.make_async_copy(
                    ct_hbm.at[:, pl.ds(0, _CH)], bc, sem_out).wait()

        @pl.when(jnp.logical_and(c < _NCHUNK, k < n_iter))
        def _():
            co = pl.multiple_of(c * _CH, _CH)
            do = pl.multiple_of(c * _CW, _CH)
            pltpu.sync_copy(wt_hbm.at[:, pl.ds(co, _CH)], bw)
            pltpu.sync_copy(ct_hbm.at[:, pl.ds(co, _CH)], bc)
            for d in range(DIM):
                pltpu.async_copy(
                    bw.at[d, :],
                    ow_hbm.at[pl.ds(do + d * _CH, _CH)], sem_out)
                pltpu.async_copy(
                    bc.at[d, :],
                    oc_hbm.at[pl.ds(do + d * _CH, _CH)], sem_out)
        return ()

    def step(k, _):
        step_slot(k, 0)
        return ()

    def step_odd(k, _):
        step_slot(k, 1)
        return ()

    def both(k2, _):
        step(2 * k2, ())
        step_odd(2 * k2 + 1, ())
        return ()

    lax.fori_loop(0, (n_iter + 2 + 1) // 2, both, ())

    @pl.when(wid == 0)
    def _():
        pltpu.sync_copy(tw_hbm, tbuf)
        pltpu.sync_copy(tbuf, ow_hbm.at[pl.ds(_TAILPOS, DIM * _NTAIL)])
        pltpu.sync_copy(tc_hbm, tbuf)
        pltpu.sync_copy(tbuf, oc_hbm.at[pl.ds(_TAILPOS, DIM * _NTAIL)])


_sc_relayout = functools.partial(
    pl.kernel,
    mesh=plsc.VectorSubcoreMesh(core_axis_name="c", subcore_axis_name="s"),
    out_type=[
        jax.ShapeDtypeStruct((_OUTN,), jnp.float32),
        jax.ShapeDtypeStruct((_OUTN,), jnp.float32),
    ],
    scratch_types=[
        pltpu.VMEM((DIM, _CH), jnp.float32),
        pltpu.VMEM((DIM, _CH), jnp.float32),
        pltpu.VMEM((DIM, _CH), jnp.float32),
        pltpu.VMEM((DIM, _CH), jnp.float32),
        pltpu.VMEM((DIM * _NTAIL,), jnp.float32),
        pltpu.SemaphoreType.DMA,
    ],
)(_sc_relayout_body)


def _sc_dot_body(i_hbm, j_hbm, ow_hbm, oc_hbm, wb_hbm, cb_hbm,
                 s_out,
                 idx_i, idx_j, base_i, step_i, base_j, step_j,
                 addr_w, addr_c, w_v, c_v, wb_v, cb_v, out_v,
                 sem_w, sem_c, sem_b):
    wid = lax.axis_index("s") * _NC + lax.axis_index("c")
    base = wid * _BPW
    pltpu.sync_copy(i_hbm.at[pl.ds(base, _BPW)], idx_i)
    pltpu.sync_copy(j_hbm.at[pl.ds(base, _BPW)], idx_j)

    cpb1 = pltpu.async_copy(wb_hbm.at[idx_i], wb_v, sem_b)
    cpb2 = pltpu.async_copy(cb_hbm.at[idx_j], cb_v, sem_b)

    def addr_prep(g, _):
        o = g * _L
        for idx, bs, st in ((idx_i, base_i, step_i),
                            (idx_j, base_j, step_j)):
            r = idx[pl.ds(o, _L)]
            tail = r >= _TAIL0
            main_b = ((lax.shift_right_logical(r, 9) * _CW)
                      + jnp.bitwise_and(r, _CH - 1))
            tail_b = _TAILPOS + (r - _TAIL0)
            bs[pl.ds(o, _L)] = jnp.where(tail, tail_b, main_b)
            st[pl.ds(o, _L)] = jnp.where(tail, _NTAIL, _CH)
        return ()

    lax.fori_loop(0, _BPW // _L, addr_prep, ())

    for d in range(DIM):
        def addr_fill(g, _):
            o = g * _L
            addr_w[d, pl.ds(o, _L)] = (base_i[pl.ds(o, _L)]
                                       + d * step_i[pl.ds(o, _L)])
            addr_c[d, pl.ds(o, _L)] = (base_j[pl.ds(o, _L)]
                                       + d * step_j[pl.ds(o, _L)])
            return ()

        lax.fori_loop(0, _BPW // _L, addr_fill, ())
        pltpu.async_copy(ow_hbm.at[addr_w.at[d, :]], w_v.at[d, :], sem_w)
        pltpu.async_copy(oc_hbm.at[addr_c.at[d, :]], c_v.at[d, :], sem_c)

    pltpu.make_async_copy(ow_hbm.at[pl.ds(0, _BPW)], w_v.at[0, :], sem_w).wait()
    for d in range(1, DIM):
        pltpu.make_async_copy(
            ow_hbm.at[pl.ds(0, _BPW)], w_v.at[d, :], sem_w).wait()
        pltpu.make_async_copy(
            oc_hbm.at[pl.ds(0, _BPW)], c_v.at[d - 1, :], sem_c).wait()
    pltpu.make_async_copy(
        oc_hbm.at[pl.ds(0, _BPW)], c_v.at[DIM - 1, :], sem_c).wait()

    cpb1.wait()
    cpb2.wait()

    def dot_group(g, _):
        o = g * _L
        acc = wb_v[pl.ds(o, _L)] + cb_v[pl.ds(o, _L)]
        for d in range(DIM):
            acc = acc + w_v[d, pl.ds(o, _L)] * c_v[d, pl.ds(o, _L)]
        out_v[pl.ds(o, _L)] = acc
        return ()

    lax.fori_loop(0, _BPW // _L, dot_group, ())
    pltpu.sync_copy(out_v, s_out.at[pl.ds(base, _BPW)])


_sc_dot = functools.partial(
    pl.kernel,
    mesh=plsc.VectorSubcoreMesh(core_axis_name="c", subcore_axis_name="s"),
    out_type=jax.ShapeDtypeStruct((B,), jnp.float32),
    scratch_types=[
        pltpu.VMEM((_BPW,), jnp.int32),
        pltpu.VMEM((_BPW,), jnp.int32),
        pltpu.VMEM((_BPW,), jnp.int32),
        pltpu.VMEM((_BPW,), jnp.int32),
        pltpu.VMEM((_BPW,), jnp.int32),
        pltpu.VMEM((_BPW,), jnp.int32),
        pltpu.VMEM((DIM, _BPW), jnp.int32),
        pltpu.VMEM((DIM, _BPW), jnp.int32),
        pltpu.VMEM((DIM, _BPW), jnp.float32),
        pltpu.VMEM((DIM, _BPW), jnp.float32),
        pltpu.VMEM((_BPW,), jnp.float32),
        pltpu.VMEM((_BPW,), jnp.float32),
        pltpu.VMEM((_BPW,), jnp.float32),
        pltpu.SemaphoreType.DMA,
        pltpu.SemaphoreType.DMA,
        pltpu.SemaphoreType.DMA,
    ],
    compiler_params=pltpu.CompilerParams(use_tc_tiling_on_sc=False),
)(_sc_dot_body)


_TC_BLK = 4096


def _tc_loss_body(x_ref, s_ref, o_ref):
    x = x_ref[...]
    f = jnp.where(x < X_MAX, (x * (1.0 / X_MAX)) ** ALPHA, jnp.float32(1.0))
    o_ref[...] = f * (s_ref[...] - jnp.log(x))


def _tc_loss(x, s):
    vec_spec = pl.BlockSpec((_TC_BLK,), lambda k: (k,))
    return pl.pallas_call(
        _tc_loss_body,
        grid=(B // _TC_BLK,),
        in_specs=[vec_spec, vec_spec],
        out_specs=vec_spec,
        out_shape=jax.ShapeDtypeStruct((B,), jnp.float32),
    )(x, s)


def kernel(x, i, j, w_table, c_table, w_bias, c_bias):
    i32 = i.astype(jnp.int32)
    j32 = j.astype(jnp.int32)
    tw = w_table[_TAIL0:].T.reshape(-1)
    tc = c_table[_TAIL0:].T.reshape(-1)
    ow, oc = _sc_relayout(w_table.T, c_table.T, tw, tc)
    s = _sc_dot(i32, j32, ow, oc, w_bias, c_bias)
    return _tc_loss(x, s)
